# baseline (speedup 1.0000x reference)
import functools

import jax
import jax.numpy as jnp
from jax import lax
from jax.experimental import pallas as pl
from jax.experimental.pallas import tpu as pltpu
from jax.experimental.pallas import tpu_sc as plsc

NUM_USERS = 100000
NUM_ITEMS = 50000
LATENT_DIM = 64
BATCH = 4096

_info = plsc.get_sparse_core_info()
_NC, _NS = _info.num_cores, _info.num_subcores
_NW = _NC * _NS
_BPW = BATCH // _NW


def _make_gather_kernel():
    mesh = plsc.VectorSubcoreMesh(core_axis_name="c", subcore_axis_name="s")

    @functools.partial(
        pl.kernel,
        mesh=mesh,
        out_type=[
            jax.ShapeDtypeStruct((BATCH, LATENT_DIM), jnp.float32),
            jax.ShapeDtypeStruct((BATCH, LATENT_DIM), jnp.float32),
        ],
        scratch_types=[
            pltpu.VMEM((_BPW,), jnp.int32),
            pltpu.VMEM((_BPW,), jnp.int32),
            pltpu.VMEM((_BPW, LATENT_DIM), jnp.float32),
            pltpu.VMEM((_BPW, LATENT_DIM), jnp.float32),
            pltpu.SemaphoreType.DMA,
            pltpu.SemaphoreType.DMA,
        ],
        compiler_params=pltpu.CompilerParams(use_tc_tiling_on_sc=False),
    )
    def gather2(u_table, i_table, u_idx, i_idx, u_out, i_out,
                u_idx_v, i_idx_v, u_rows_v, i_rows_v, u_sem, i_sem):
        wid = lax.axis_index("s") * _NC + lax.axis_index("c")
        base = wid * _BPW
        pltpu.sync_copy(u_idx.at[pl.ds(base, _BPW)], u_idx_v)
        pltpu.sync_copy(i_idx.at[pl.ds(base, _BPW)], i_idx_v)
        cu = pltpu.async_copy(u_table.at[u_idx_v], u_rows_v, u_sem)
        ci = pltpu.async_copy(i_table.at[i_idx_v], i_rows_v, i_sem)
        cu.wait()
        pltpu.sync_copy(u_rows_v, u_out.at[pl.ds(base, _BPW)])
        ci.wait()
        pltpu.sync_copy(i_rows_v, i_out.at[pl.ds(base, _BPW)])

    return gather2


_gather2 = _make_gather_kernel()


def kernel(user_indices, item_indices, user_embeddings, item_embeddings):
    u_idx = user_indices.astype(jnp.int32)
    i_idx = item_indices.astype(jnp.int32)
    u_out, i_out = _gather2(user_embeddings, item_embeddings, u_idx, i_idx)
    return (u_out, i_out)


# speedup vs baseline: 2.2541x; 2.2541x over previous
import functools

import jax
import jax.numpy as jnp
from jax import lax
from jax.experimental import pallas as pl
from jax.experimental.pallas import tpu as pltpu
from jax.experimental.pallas import tpu_sc as plsc

NUM_USERS = 100000
NUM_ITEMS = 50000
LATENT_DIM = 64
BATCH = 4096

_info = plsc.get_sparse_core_info()
_NC, _NS, _NL = _info.num_cores, _info.num_subcores, _info.num_lanes

_UCH = -(-NUM_USERS // 128)
_ICH = -(-NUM_ITEMS // 128)
_HALF = _ICH * 128
_OCH = BATCH // 128


def _make_gather_kernel():
    mesh = plsc.VectorSubcoreMesh(core_axis_name="c", subcore_axis_name="s")

    @functools.partial(
        pl.kernel,
        mesh=mesh,
        out_type=[
            jax.ShapeDtypeStruct((LATENT_DIM, BATCH), jnp.float32),
            jax.ShapeDtypeStruct((LATENT_DIM, BATCH), jnp.float32),
        ],
        scratch_types=[
            pltpu.VMEM((2, _HALF), jnp.float32),
            pltpu.VMEM((BATCH,), jnp.int32),
            pltpu.VMEM((BATCH,), jnp.int32),
            pltpu.VMEM((2, BATCH), jnp.float32),
            pltpu.VMEM((2, BATCH), jnp.float32),
            pltpu.SemaphoreType.DMA,
            pltpu.SemaphoreType.DMA,
        ],
        compiler_params=pltpu.CompilerParams(needs_layout_passes=False),
    )
    def gather2(u_t, i_t, u_tail, i_tail, u_idx, i_idx, u_out, i_out,
                colab, u_idx_v, i_idx_v, ob_u, ob_i, sem_in, sem_out):
        cid = lax.axis_index("c")
        sid = lax.axis_index("s")
        col0 = cid * (LATENT_DIM // 2) + sid * 2

        pltpu.sync_copy(u_idx.at[:], u_idx_v)
        pltpu.sync_copy(i_idx.at[:], i_idx_v)

        def fire_chunks(table, rb_lo, rb_hi, local0):
            def body(rb, carry):
                src0 = pl.multiple_of(128 * rb, 128)
                dst0 = pl.multiple_of(128 * rb - local0, 128)
                pltpu.async_copy(
                    table.at[pl.ds(col0, 2), pl.ds(src0, 128)],
                    colab.at[:, pl.ds(dst0, 128)], sem_in)
                return carry
            lax.fori_loop(rb_lo, rb_hi, body, 0)

        def fire_tail(tail, n_rows, local0):
            pltpu.async_copy(
                tail.at[pl.ds(col0, 2), :],
                colab.at[:, pl.ds(n_rows // 128 * 128 - local0, 128)],
                sem_in)

        def drain_in():
            pltpu.make_async_copy(
                u_t.at[pl.ds(0, 2), pl.ds(0, _HALF)], colab, sem_in).wait()

        def gather_pass(idx_v, ob, local0, extent, merge):
            def body(i, carry):
                iv = idx_v[pl.ds(i * _NL, _NL)]
                loc = iv - local0
                for k in range(2):
                    kvec = jnp.full((_NL,), k, dtype=jnp.int32)
                    if merge is None:
                        ob[k, pl.ds(i * _NL, _NL)] = plsc.load_gather(
                            colab, [kvec, loc])
                    else:
                        m = (loc >= 0) & (loc < extent)
                        val = plsc.load_gather(colab, [kvec, loc], mask=m)
                        if merge == "init":
                            ob[k, pl.ds(i * _NL, _NL)] = jnp.where(
                                m, val, jnp.float32(0))
                        else:
                            prev = ob[k, pl.ds(i * _NL, _NL)]
                            ob[k, pl.ds(i * _NL, _NL)] = jnp.where(
                                m, val, prev)
                return carry
            lax.fori_loop(0, BATCH // _NL, body, 0, unroll=4)

        def fire_out(ob, out_hbm):
            def body(rb, carry):
                o0 = pl.multiple_of(128 * rb, 128)
                pltpu.async_copy(
                    ob.at[:, pl.ds(o0, 128)],
                    out_hbm.at[pl.ds(col0, 2), pl.ds(o0, 128)],
                    sem_out)
                return carry
            lax.fori_loop(0, _OCH, body, 0)

        fire_chunks(u_t, 0, _ICH, 0)
        drain_in()
        gather_pass(u_idx_v, ob_u, 0, _HALF, "init")
        fire_chunks(u_t, _ICH, NUM_USERS // 128, _HALF)
        fire_tail(u_tail, NUM_USERS, _HALF)
        drain_in()
        gather_pass(u_idx_v, ob_u, _HALF, NUM_USERS - _HALF, "merge")
        fire_out(ob_u, u_out)
        fire_chunks(i_t, 0, NUM_ITEMS // 128, 0)
        fire_tail(i_tail, NUM_ITEMS, 0)
        drain_in()
        gather_pass(i_idx_v, ob_i, 0, NUM_ITEMS, None)
        fire_out(ob_i, i_out)
        pltpu.make_async_copy(u_t.at[pl.ds(0, 2), pl.ds(0, BATCH)],
                              ob_u, sem_out).wait()
        pltpu.make_async_copy(u_t.at[pl.ds(0, 2), pl.ds(0, BATCH)],
                              ob_i, sem_out).wait()

    return gather2


_gather2 = _make_gather_kernel()


def kernel(user_indices, item_indices, user_embeddings, item_embeddings):
    u_idx = user_indices.astype(jnp.int32)
    i_idx = item_indices.astype(jnp.int32)
    u_tail = jnp.pad(user_embeddings[NUM_USERS // 128 * 128:, :],
                     ((0, 128 - NUM_USERS % 128), (0, 0))).T
    i_tail = jnp.pad(item_embeddings[NUM_ITEMS // 128 * 128:, :],
                     ((0, 128 - NUM_ITEMS % 128), (0, 0))).T
    u_out_t, i_out_t = _gather2(user_embeddings.T, item_embeddings.T,
                                u_tail, i_tail, u_idx, i_idx)
    return (u_out_t.T, i_out_t.T)
